# Initial kernel scaffold; baseline (speedup 1.0000x reference)
import jax
import jax.numpy as jnp
from jax.experimental import pallas as pl


def kernel(x, edge_index, W1, b1, W2, b2, pool_w, W3, b3, W4, b4, Wl, bl):
    raise NotImplementedError("write your pallas kernel here")


# speedup vs baseline: 9.5793x; 9.5793x over previous
import functools

import jax
import jax.numpy as jnp
from jax import lax
from jax.experimental import pallas as pl
from jax.experimental.pallas import tpu as pltpu
from jax.experimental.pallas import tpu_sc as plsc

N = 10000
E = 320000
D = 128
H = 64
C = 10
K = 5000
NPAD = 10240
ACC2 = NPAD + 1024
KPAD = 5120
NW = 32
CH = 128
TPW = 80
EP = NW * TPW * CH
PF = 2


def _mesh():
    return plsc.VectorSubcoreMesh(core_axis_name="c", subcore_axis_name="s",
                                  num_cores=2, num_subcores=16)


@functools.partial(
    pl.kernel,
    out_type=jax.ShapeDtypeStruct((2, NPAD, D), jnp.float32),
    mesh=_mesh(),
    scratch_types=[
        pltpu.MemorySpace.VMEM_SHARED((NPAD, D), jnp.float32),
        pltpu.VMEM((2, 2, CH), jnp.int32),
        pltpu.VMEM((2, CH, D), jnp.float32),
        pltpu.SemaphoreType.DMA,
        pltpu.SemaphoreType.DMA,
        pltpu.SemaphoreType.DMA,
    ],
)
def _sc_edge_agg1(x_hbm, sd_hbm, zero_hbm, out_hbm,
                  acc, sd_v, rows_v, gsem, isem0, isem1):
    c = lax.axis_index("c")
    s = lax.axis_index("s")
    wid = s * 2 + c
    rpt = NPAD // 16
    isem = (isem0, isem1)
    pltpu.sync_copy(zero_hbm.at[pl.ds(s * rpt, rpt)],
                    acc.at[pl.ds(s * rpt, rpt)])
    plsc.subcore_barrier()

    pltpu.async_copy(sd_hbm.at[wid, 0], sd_v.at[0], isem0).wait()
    pltpu.async_copy(x_hbm.at[sd_v.at[0, 0]], rows_v.at[0], gsem)
    pltpu.async_copy(sd_hbm.at[wid, 1], sd_v.at[1], isem1)

    def body(g, carry):
        for b in (0, 1):
            t = 2 * g + b
            nb = 1 - b
            pltpu.make_async_copy(x_hbm.at[sd_v.at[b, 0]], rows_v.at[b],
                                  gsem).wait()
            pltpu.make_async_copy(sd_hbm.at[wid, t + 1], sd_v.at[nb],
                                  isem[nb]).wait()
            pltpu.async_copy(x_hbm.at[sd_v.at[nb, 0]], rows_v.at[nb], gsem)
            pltpu.sync_copy(rows_v.at[b], acc.at[sd_v.at[b, 1]], add=True)
            pltpu.async_copy(sd_hbm.at[wid, t + 2], sd_v.at[b], isem[b])
        return carry

    lax.fori_loop(0, TPW // 2, body, 0)
    pltpu.make_async_copy(x_hbm.at[sd_v.at[0, 0]], rows_v.at[0], gsem).wait()
    pltpu.make_async_copy(sd_hbm.at[wid, TPW + 1], sd_v.at[1], isem1).wait()
    plsc.subcore_barrier()
    pltpu.sync_copy(acc.at[pl.ds(s * rpt, rpt)],
                    out_hbm.at[c, pl.ds(s * rpt, rpt)])


@functools.partial(
    pl.kernel,
    out_type=jax.ShapeDtypeStruct((2, NPAD, H), jnp.float32),
    mesh=_mesh(),
    scratch_types=[
        pltpu.MemorySpace.VMEM_SHARED((ACC2, H), jnp.float32),
        pltpu.VMEM((NPAD // 128, 128), jnp.int32),
        pltpu.VMEM((2, 2, CH), jnp.int32),
        pltpu.VMEM((2, CH), jnp.int32),
        pltpu.VMEM((2, CH, H), jnp.float32),
        pltpu.SemaphoreType.DMA,
        pltpu.SemaphoreType.DMA,
        pltpu.SemaphoreType.DMA,
    ],
    compiler_params=pltpu.CompilerParams(needs_layout_passes=False,
                                         use_tc_tiling_on_sc=False),
)
def _sc_edge_agg2(outs_hbm, sd_hbm, rank_hbm, zero_hbm, out_hbm,
                  acc, rank_v, sd_v, dstm_v, rows_v, gsem, isem0, isem1):
    c = lax.axis_index("c")
    s = lax.axis_index("s")
    wid = s * 2 + c
    zpt = ACC2 // 16
    isem = (isem0, isem1)
    pltpu.sync_copy(zero_hbm.at[pl.ds(s * zpt, zpt)],
                    acc.at[pl.ds(s * zpt, zpt)])
    pltpu.sync_copy(rank_hbm, rank_v)
    plsc.subcore_barrier()

    def masked_dst(b):
        for j in range(CH // 16):
            s16 = sd_v[b, 0, pl.ds(j * 16, 16)]
            d16 = sd_v[b, 1, pl.ds(j * 16, 16)]
            rs = plsc.load_gather(rank_v, [s16 >> 7, s16 & 127])
            rd = plsc.load_gather(rank_v, [d16 >> 7, d16 & 127])
            kept = (rs < K) & (rd < K)
            dummy = NPAD + ((s16 + d16) & 1023)
            dstm_v[b, pl.ds(j * 16, 16)] = jnp.where(kept, d16, dummy)

    pltpu.async_copy(sd_hbm.at[wid, 0], sd_v.at[0], isem0).wait()
    pltpu.async_copy(outs_hbm.at[sd_v.at[0, 0]], rows_v.at[0], gsem)
    pltpu.async_copy(sd_hbm.at[wid, 1], sd_v.at[1], isem1)

    def body(g, carry):
        for b in (0, 1):
            t = 2 * g + b
            nb = 1 - b
            masked_dst(b)
            pltpu.make_async_copy(outs_hbm.at[sd_v.at[b, 0]], rows_v.at[b],
                                  gsem).wait()
            pltpu.make_async_copy(sd_hbm.at[wid, t + 1], sd_v.at[nb],
                                  isem[nb]).wait()
            pltpu.async_copy(outs_hbm.at[sd_v.at[nb, 0]], rows_v.at[nb], gsem)
            pltpu.sync_copy(rows_v.at[b], acc.at[dstm_v.at[b]], add=True)
            pltpu.async_copy(sd_hbm.at[wid, t + 2], sd_v.at[b], isem[b])
        return carry

    lax.fori_loop(0, TPW // 2, body, 0)
    pltpu.make_async_copy(outs_hbm.at[sd_v.at[0, 0]], rows_v.at[0],
                          gsem).wait()
    pltpu.make_async_copy(sd_hbm.at[wid, TPW + 1], sd_v.at[1], isem1).wait()
    plsc.subcore_barrier()
    rpt = NPAD // 16
    pltpu.sync_copy(acc.at[pl.ds(s * rpt, rpt)],
                    out_hbm.at[c, pl.ds(s * rpt, rpt)])


@functools.partial(
    pl.kernel,
    out_type=jax.ShapeDtypeStruct((KPAD, 16), jnp.float32),
    mesh=_mesh(),
    scratch_types=[
        pltpu.VMEM((NPAD // 128, 128), jnp.int32),
        pltpu.VMEM((KPAD // CH, CH), jnp.int32),
        pltpu.VMEM((KPAD, 16), jnp.float32),
        pltpu.SemaphoreType.DMA,
    ],
    compiler_params=pltpu.CompilerParams(needs_layout_passes=False,
                                         use_tc_tiling_on_sc=False),
)
def _sc_permute(rank_hbm, y_hbm, out_hbm, rank_v, perm_v, y_v, sem):
    c = lax.axis_index("c")
    s = lax.axis_index("s")
    wid = s * 2 + c

    @pl.when(wid == 0)
    def _():
        pltpu.sync_copy(rank_hbm, rank_v)
        zero16 = jnp.zeros((16,), jnp.int32)
        for a in range(KPAD // CH):
            for j in range(CH // 16):
                perm_v[a, pl.ds(j * 16, 16)] = zero16

        def body(a, carry):
            for j in range(128 // 16):
                r16 = rank_v[a, pl.ds(j * 16, 16)]
                i16 = lax.iota(jnp.int32, 16) + a * 128 + j * 16
                plsc.store_scatter(perm_v, [r16 >> 7, r16 & 127], i16,
                                   mask=r16 < K)
            return carry

        lax.fori_loop(0, NPAD // 128, body, 0)
        cps = [pltpu.async_copy(y_hbm.at[perm_v.at[g]],
                                y_v.at[pl.ds(g * CH, CH)], sem)
               for g in range(KPAD // CH)]
        for cp in cps:
            cp.wait()
        pltpu.sync_copy(y_v, out_hbm)


def _tc_rank(scol, srow):
    IB = 1024
    JC = 2048

    def body(scol_ref, srow_ref, o_ref):
        p = pl.program_id(0)
        i0 = p * IB
        si = scol_ref[...]
        cnt = jnp.zeros((IB, 1), jnp.float32)
        for cch in range(NPAD // JC):
            sj = srow_ref[:, pl.ds(cch * JC, JC)]

            def diag_fn():
                gt = (sj > si).astype(jnp.float32)
                eq = sj == si
                ig = i0 + lax.broadcasted_iota(jnp.int32, (IB, JC), 0)
                jg = cch * JC + lax.broadcasted_iota(jnp.int32, (IB, JC), 1)
                return gt + jnp.where(eq & (jg < ig), 1.0, 0.0)

            def below_fn():
                return (sj >= si).astype(jnp.float32)

            def above_fn():
                return (sj > si).astype(jnp.float32)

            contrib = lax.cond(
                (i0 // JC) == cch, diag_fn,
                lambda: lax.cond(cch * JC + JC <= i0, below_fn, above_fn))
            cnt = cnt + jnp.sum(contrib, axis=1, keepdims=True)
        o_ref[...] = cnt.astype(jnp.int32)

    return pl.pallas_call(
        body,
        grid=(NPAD // IB,),
        in_specs=[pl.BlockSpec((IB, 1), lambda p: (p, 0)),
                  pl.BlockSpec((1, NPAD), lambda p: (0, 0))],
        out_specs=pl.BlockSpec((IB, 1), lambda p: (p, 0)),
        out_shape=jax.ShapeDtypeStruct((NPAD, 1), jnp.int32),
    )(scol, srow)


def _tc_mlp1(xp, agg, scol, W1, b1, W2, b2):
    RB = 2048

    def body(x_ref, a_ref, sc_ref, w1_ref, b1_ref, w2_ref, b2_ref, o_ref):
        z = x_ref[...] + a_ref[0] + a_ref[1]
        h = jnp.maximum(
            jnp.dot(z, w1_ref[...], preferred_element_type=jnp.float32)
            + b1_ref[...], 0.0)
        h = jnp.dot(h, w2_ref[...], preferred_element_type=jnp.float32) \
            + b2_ref[...]
        o_ref[...] = jnp.maximum(h, 0.0) * sc_ref[...]

    return pl.pallas_call(
        body,
        grid=(NPAD // RB,),
        in_specs=[pl.BlockSpec((RB, D), lambda p: (p, 0)),
                  pl.BlockSpec((2, RB, D), lambda p: (0, p, 0)),
                  pl.BlockSpec((RB, 1), lambda p: (p, 0)),
                  pl.BlockSpec((D, H), lambda p: (0, 0)),
                  pl.BlockSpec((1, H), lambda p: (0, 0)),
                  pl.BlockSpec((H, H), lambda p: (0, 0)),
                  pl.BlockSpec((1, H), lambda p: (0, 0))],
        out_specs=pl.BlockSpec((RB, H), lambda p: (p, 0)),
        out_shape=jax.ShapeDtypeStruct((NPAD, H), jnp.float32),
    )(xp, agg, scol, W1, b1, W2, b2)


def _tc_mlp2(outs, acc2, W3, b3, W4, b4, Wlp, blp):
    RB = 2048

    def body(o_in_ref, a_ref, w3_ref, b3_ref, w4_ref, b4_ref,
             wl_ref, bl_ref, y_ref):
        z = o_in_ref[...] + a_ref[0] + a_ref[1]
        h = jnp.maximum(
            jnp.dot(z, w3_ref[...], preferred_element_type=jnp.float32)
            + b3_ref[...], 0.0)
        h = jnp.dot(h, w4_ref[...], preferred_element_type=jnp.float32) \
            + b4_ref[...]
        h = jnp.maximum(h, 0.0)
        y_ref[...] = jnp.dot(h, wl_ref[...],
                             preferred_element_type=jnp.float32) + bl_ref[...]

    return pl.pallas_call(
        body,
        grid=(NPAD // RB,),
        in_specs=[pl.BlockSpec((RB, H), lambda p: (p, 0)),
                  pl.BlockSpec((2, RB, H), lambda p: (0, p, 0)),
                  pl.BlockSpec((H, H), lambda p: (0, 0)),
                  pl.BlockSpec((1, H), lambda p: (0, 0)),
                  pl.BlockSpec((H, H), lambda p: (0, 0)),
                  pl.BlockSpec((1, H), lambda p: (0, 0)),
                  pl.BlockSpec((H, 16), lambda p: (0, 0)),
                  pl.BlockSpec((1, 16), lambda p: (0, 0))],
        out_specs=pl.BlockSpec((RB, 16), lambda p: (p, 0)),
        out_shape=jax.ShapeDtypeStruct((NPAD, 16), jnp.float32),
    )(outs, acc2, W3, b3, W4, b4, Wlp, blp)


def kernel(x, edge_index, W1, b1, W2, b2, pool_w, W3, b3, W4, b4, Wl, bl):
    f32 = jnp.float32
    src = edge_index[0].astype(jnp.int32)
    dst = edge_index[1].astype(jnp.int32)
    pad = EP - E
    ppos = jnp.arange(pad, dtype=jnp.int32)
    src_p = jnp.concatenate([src, (ppos * 97) % N]).reshape(NW, TPW, CH)
    dst_p = jnp.concatenate([dst, N + (ppos % 64)]).reshape(NW, TPW, CH)
    qpos = jnp.arange(NW * PF * CH, dtype=jnp.int32)
    ext_s = ((qpos * 89) % N).reshape(NW, PF, CH)
    ext_d = (N + (qpos % 64)).reshape(NW, PF, CH)
    sd = jnp.stack([jnp.concatenate([src_p, ext_s], axis=1),
                    jnp.concatenate([dst_p, ext_d], axis=1)], axis=2)
    xp = jnp.zeros((NPAD, D), f32).at[:N].set(x)
    zero1 = jnp.zeros((NPAD, D), f32)
    zero2 = jnp.zeros((ACC2, H), f32)

    agg = _sc_edge_agg1(xp, sd, zero1)

    score = jnp.tanh((x * pool_w[None, :]).sum(-1) / jnp.linalg.norm(pool_w))
    spad = jnp.full((NPAD,), -2.0, f32).at[:N].set(score)
    scol = spad.reshape(NPAD, 1)
    srow = spad.reshape(1, NPAD)
    rankc = _tc_rank(scol, srow)
    rank2d = rankc.reshape(NPAD // 128, 128)

    outs = _tc_mlp1(xp, agg, scol, W1, b1.reshape(1, H), W2, b2.reshape(1, H))
    acc2 = _sc_edge_agg2(outs, sd, rank2d, zero2)

    Wlp = jnp.zeros((H, 16), f32).at[:, :C].set(Wl)
    blp = jnp.zeros((1, 16), f32).at[0, :C].set(bl)
    y = _tc_mlp2(outs, acc2, W3, b3.reshape(1, H), W4, b4.reshape(1, H),
                 Wlp, blp)
    yp = _sc_permute(rank2d, y)
    return yp[:K, :C]
